# baseline (speedup 1.0000x reference)
import functools

import jax
import jax.numpy as jnp
import numpy as np
from jax import lax
from jax.experimental import pallas as pl
from jax.experimental.pallas import tpu as pltpu
from jax.experimental.pallas import tpu_sc as plsc

_NC = 2
_NS = 16
_NW = _NC * _NS

_CHUNK = 64

_PERM_CACHE = {}


def _tf2x32(k1, k2, x0, x1):
    rot = ((13, 15, 26, 6), (17, 29, 16, 24))
    ks = (np.uint32(k1), np.uint32(k2),
          np.uint32(k1) ^ np.uint32(k2) ^ np.uint32(0x1BD11BDA))
    x0 = (x0 + ks[0]).astype(np.uint32)
    x1 = (x1 + ks[1]).astype(np.uint32)
    for i in range(5):
        for r in rot[i % 2]:
            x0 = (x0 + x1).astype(np.uint32)
            x1 = ((x1 << np.uint32(r)) | (x1 >> np.uint32(32 - r))).astype(np.uint32)
            x1 = x0 ^ x1
        x0 = (x0 + ks[(i + 1) % 3]).astype(np.uint32)
        x1 = (x1 + ks[(i + 2) % 3] + np.uint32(i + 1)).astype(np.uint32)
    return x0, x1


def _perm_const(n: int) -> np.ndarray:
    if n not in _PERM_CACHE:
        key = (np.uint32(0), np.uint32(42))
        x = np.arange(n, dtype=np.int32)
        num_rounds = int(np.ceil(3 * np.log(max(1, n)) / np.log(2**32 - 1)))
        for _ in range(num_rounds):
            b1, b2 = _tf2x32(key[0], key[1], np.zeros(2, np.uint32),
                             np.arange(2, dtype=np.uint32))
            key, subkey = (b1[0], b2[0]), (b1[1], b2[1])
            s1, s2 = _tf2x32(subkey[0], subkey[1], np.zeros(n, np.uint32),
                             np.arange(n, dtype=np.uint32))
            x = x[np.argsort(s1 ^ s2, kind="stable")]
        _PERM_CACHE[n] = x
    return _PERM_CACHE[n]


def _make_gather(n: int, d: int):
    chunk = 80
    nch_w = 42
    span = chunk * nch_w
    assert span * _NW >= n and span % 8 == 0 and chunk % 8 == 0
    mesh = plsc.VectorSubcoreMesh(core_axis_name="c", subcore_axis_name="s")

    @functools.partial(
        pl.kernel,
        mesh=mesh,
        out_type=jax.ShapeDtypeStruct((n, d), jnp.float32),
        scratch_types=[
            pltpu.VMEM((span,), jnp.int32),
            pltpu.VMEM((chunk, d), jnp.float32),
            pltpu.VMEM((chunk, d), jnp.float32),
            pltpu.VMEM((chunk, d), jnp.float32),
            pltpu.SemaphoreType.DMA,
            pltpu.SemaphoreType.DMA,
            pltpu.SemaphoreType.DMA,
            pltpu.SemaphoreType.DMA,
            pltpu.SemaphoreType.DMA,
            pltpu.SemaphoreType.DMA,
        ],
    )
    def shuffle_k(x_hbm, perm_hbm, out_hbm, idx_v, r0, r1, r2,
                  g0, g1, g2, w0, w1, w2):
        wid = lax.axis_index("s") * _NC + lax.axis_index("c")
        rows = (r0, r1, r2)
        gsems = (g0, g1, g2)
        wsems = (w0, w1, w2)
        base_w = jnp.minimum(wid * span, n - span)
        base_w = pl.multiple_of(base_w, 8)
        pltpu.sync_copy(perm_hbm.at[pl.ds(base_w, span)], idx_v)

        def start_gather(j, b):
            off = pl.multiple_of(j * chunk, 8)
            pltpu.async_copy(x_hbm.at[idx_v.at[pl.ds(off, chunk)]],
                             rows[b], gsems[b])

        def wait_gather(b):
            pltpu.make_async_copy(x_hbm.at[pl.ds(0, chunk)], rows[b],
                                  gsems[b]).wait()

        def start_write(j, b):
            off = pl.multiple_of(base_w + j * chunk, 8)
            pltpu.async_copy(rows[b], out_hbm.at[pl.ds(off, chunk)], wsems[b])

        def wait_write(b):
            pltpu.make_async_copy(rows[b], out_hbm.at[pl.ds(0, chunk)],
                                  wsems[b]).wait()

        start_gather(0, 0)
        start_gather(1, 1)
        start_gather(2, 2)

        def body(r, carry):
            for bb in range(3):
                k = r * 3 + bb
                bp = (bb + 2) % 3
                wait_gather(bb)
                start_write(k, bb)

                @pl.when(k >= 1)
                def _():
                    wait_write(bp)

                @pl.when((k >= 1) & (k + 2 < nch_w))
                def _():
                    start_gather(k + 2, bp)

            return carry

        lax.fori_loop(0, nch_w // 3, body, 0)
        wait_write((nch_w - 1) % 3)

    return shuffle_k


def _make_scatter(n: int, d: int):
    chunk = 112
    nch_w = 28
    span = chunk * nch_w
    assert span * _NW >= n and span % 8 == 0 and chunk % 8 == 0
    mesh = plsc.VectorSubcoreMesh(core_axis_name="c", subcore_axis_name="s")

    @functools.partial(
        pl.kernel,
        mesh=mesh,
        out_type=jax.ShapeDtypeStruct((n, d), jnp.float32),
        scratch_types=[
            pltpu.VMEM((chunk,), jnp.int32),
            pltpu.VMEM((chunk,), jnp.int32),
            pltpu.VMEM((chunk, d), jnp.float32),
            pltpu.VMEM((chunk, d), jnp.float32),
            pltpu.SemaphoreType.DMA,
            pltpu.SemaphoreType.DMA,
            pltpu.SemaphoreType.DMA,
            pltpu.SemaphoreType.DMA,
        ],
    )
    def shuffle_k(x_hbm, inv_hbm, out_hbm, i0, i1, r0, r1, g0, g1, s0, s1):
        wid = lax.axis_index("s") * _NC + lax.axis_index("c")
        idxs = (i0, i1)
        rows = (r0, r1)
        gsems = (g0, g1)
        ssems = (s0, s1)
        base_w = jnp.minimum(wid * span, n - span)
        base_w = pl.multiple_of(base_w, 8)

        def start_load(j, b):
            off = pl.multiple_of(base_w + j * chunk, 8)
            pltpu.async_copy(x_hbm.at[pl.ds(off, chunk)], rows[b], gsems[b])
            pltpu.async_copy(inv_hbm.at[pl.ds(off, chunk)], idxs[b], gsems[b])

        def wait_load(b):
            pltpu.make_async_copy(x_hbm.at[pl.ds(0, chunk)], rows[b],
                                  gsems[b]).wait()
            pltpu.make_async_copy(inv_hbm.at[pl.ds(0, chunk)], idxs[b],
                                  gsems[b]).wait()

        start_load(0, 0)
        start_load(1, 1)

        def body(jj, carry):
            for b in range(2):
                j = jj * 2 + b
                wait_load(b)
                pltpu.async_copy(rows[b], out_hbm.at[idxs[b]], ssems[b])
                pltpu.make_async_copy(rows[b], out_hbm.at[pl.ds(0, chunk)],
                                      ssems[b]).wait()

                @pl.when(j + 2 < nch_w)
                def _():
                    start_load(j + 2, b)

            return carry

        lax.fori_loop(0, nch_w // 2, body, 0)

    return shuffle_k


def _make_hybrid(n: int, d: int):
    h = d // 2
    chunk = 112
    nch_w = 28
    span = chunk * nch_w
    assert span * _NW >= n and span % 8 == 0 and chunk % 8 == 0
    mesh = plsc.VectorSubcoreMesh(core_axis_name="c", subcore_axis_name="s")

    @functools.partial(
        pl.kernel,
        mesh=mesh,
        out_type=jax.ShapeDtypeStruct((n, 2, h), jnp.float32),
        scratch_types=[
            pltpu.VMEM((chunk,), jnp.int32),
            pltpu.VMEM((chunk,), jnp.int32),
            pltpu.VMEM((chunk,), jnp.int32),
            pltpu.VMEM((chunk,), jnp.int32),
            pltpu.VMEM((chunk, 1, h), jnp.float32),
            pltpu.VMEM((chunk, 1, h), jnp.float32),
            pltpu.VMEM((chunk, 1, h), jnp.float32),
            pltpu.VMEM((chunk, 1, h), jnp.float32),
            pltpu.SemaphoreType.DMA,
            pltpu.SemaphoreType.DMA,
            pltpu.SemaphoreType.DMA,
            pltpu.SemaphoreType.DMA,
            pltpu.SemaphoreType.DMA,
            pltpu.SemaphoreType.DMA,
        ],
    )
    def shuffle_k(x_hbm, perm_hbm, inv_hbm, out_hbm,
                  ia0, ia1, ib0, ib1, ra0, ra1, rb0, rb1,
                  ga0, ga1, gb0, gb1, sb0, sb1):
        wid = lax.axis_index("s") * _NC + lax.axis_index("c")
        idxa = (ia0, ia1)
        idxb = (ib0, ib1)
        rowsa = (ra0, ra1)
        rowsb = (rb0, rb1)
        gasems = (ga0, ga1)
        gbsems = (gb0, gb1)
        sbsems = (sb0, sb1)
        base_w = jnp.minimum(wid * span, n - span)
        base_w = pl.multiple_of(base_w, 8)

        def start_load_a(j, b):
            off = pl.multiple_of(base_w + j * chunk, 8)
            pltpu.sync_copy(perm_hbm.at[pl.ds(off, chunk)], idxa[b])
            pltpu.async_copy(x_hbm.at[idxa[b], pl.ds(0, 1)], rowsa[b],
                             gasems[b])

        def wait_load_a(b):
            pltpu.make_async_copy(x_hbm.at[pl.ds(0, chunk), pl.ds(0, 1)],
                                  rowsa[b], gasems[b]).wait()

        def start_load_b(j, b):
            off = pl.multiple_of(base_w + j * chunk, 8)
            pltpu.sync_copy(inv_hbm.at[pl.ds(off, chunk)], idxb[b])
            pltpu.async_copy(x_hbm.at[pl.ds(off, chunk), pl.ds(1, 1)],
                             rowsb[b], gbsems[b])

        def wait_load_b(b):
            pltpu.make_async_copy(x_hbm.at[pl.ds(0, chunk), pl.ds(1, 1)],
                                  rowsb[b], gbsems[b]).wait()

        def wait_scatter_b(b):
            pltpu.make_async_copy(rowsb[b],
                                  out_hbm.at[pl.ds(0, chunk), pl.ds(1, 1)],
                                  sbsems[b]).wait()

        start_load_a(0, 0)
        start_load_b(0, 0)
        start_load_a(1, 1)
        start_load_b(1, 1)

        def body(jj, carry):
            for b in range(2):
                j = jj * 2 + b
                off = pl.multiple_of(base_w + j * chunk, 8)
                wait_load_b(b)
                pltpu.async_copy(rowsb[b], out_hbm.at[idxb[b], pl.ds(1, 1)],
                                 sbsems[b])
                wait_load_a(b)
                pltpu.sync_copy(rowsa[b],
                                out_hbm.at[pl.ds(off, chunk), pl.ds(0, 1)])
                wait_scatter_b(b)

                @pl.when(j + 2 < nch_w)
                def _():
                    start_load_a(j + 2, b)
                    start_load_b(j + 2, b)

            return carry

        lax.fori_loop(0, nch_w // 2, body, 0)

    return shuffle_k


def kernel(X):
    n, d = X.shape
    perm = _perm_const(n)
    inv = jnp.asarray(np.argsort(perm).astype(np.int32))
    x3 = X.reshape(n, 2, d // 2)
    out3 = _make_hybrid(n, d)(x3, jnp.asarray(perm), inv)
    return out3.reshape(n, d)


# speedup vs baseline: 3.4371x; 3.4371x over previous
import functools

import jax
import jax.numpy as jnp
import numpy as np
from jax import lax
from jax.experimental import pallas as pl
from jax.experimental.pallas import tpu as pltpu
from jax.experimental.pallas import tpu_sc as plsc

_NC = 2
_NS = 16
_NW = _NC * _NS

_PERM_CACHE = {}


def _tf2x32(k1, k2, x0, x1):
    rot = ((13, 15, 26, 6), (17, 29, 16, 24))
    ks = (np.uint32(k1), np.uint32(k2),
          np.uint32(k1) ^ np.uint32(k2) ^ np.uint32(0x1BD11BDA))
    x0 = (x0 + ks[0]).astype(np.uint32)
    x1 = (x1 + ks[1]).astype(np.uint32)
    for i in range(5):
        for r in rot[i % 2]:
            x0 = (x0 + x1).astype(np.uint32)
            x1 = ((x1 << np.uint32(r)) | (x1 >> np.uint32(32 - r))).astype(np.uint32)
            x1 = x0 ^ x1
        x0 = (x0 + ks[(i + 1) % 3]).astype(np.uint32)
        x1 = (x1 + ks[(i + 2) % 3] + np.uint32(i + 1)).astype(np.uint32)
    return x0, x1


def _perm_const(n: int) -> np.ndarray:
    if n not in _PERM_CACHE:
        key = (np.uint32(0), np.uint32(42))
        x = np.arange(n, dtype=np.int32)
        num_rounds = int(np.ceil(3 * np.log(max(1, n)) / np.log(2**32 - 1)))
        for _ in range(num_rounds):
            b1, b2 = _tf2x32(key[0], key[1], np.zeros(2, np.uint32),
                             np.arange(2, dtype=np.uint32))
            key, subkey = (b1[0], b2[0]), (b1[1], b2[1])
            s1, s2 = _tf2x32(subkey[0], subkey[1], np.zeros(n, np.uint32),
                             np.arange(n, dtype=np.uint32))
            x = x[np.argsort(s1 ^ s2, kind="stable")]
        _PERM_CACHE[n] = x
    return _PERM_CACHE[n]


def _make_gather(n: int, d: int):
    chunk = 112
    nch_w = 28
    span = chunk * nch_w
    assert span * _NW >= n and span % 8 == 0 and chunk % 8 == 0
    mesh = plsc.VectorSubcoreMesh(core_axis_name="c", subcore_axis_name="s")

    @functools.partial(
        pl.kernel,
        mesh=mesh,
        out_type=jax.ShapeDtypeStruct((n, d), jnp.float32),
        scratch_types=[
            pltpu.VMEM((span,), jnp.int32),
            pltpu.VMEM((chunk, d), jnp.float32),
            pltpu.VMEM((chunk, d), jnp.float32),
            pltpu.SemaphoreType.DMA,
            pltpu.SemaphoreType.DMA,
        ],
    )
    def shuffle_k(x_hbm, perm_hbm, out_hbm, idx_v, rows0, rows1, sem0, sem1):
        wid = lax.axis_index("s") * _NC + lax.axis_index("c")
        rows = (rows0, rows1)
        sems = (sem0, sem1)
        base_w = jnp.minimum(wid * span, n - span)
        base_w = pl.multiple_of(base_w, 8)
        pltpu.sync_copy(perm_hbm.at[pl.ds(base_w, span)], idx_v)

        def start_gather(j, b):
            off = pl.multiple_of(j * chunk, 8)
            pltpu.async_copy(x_hbm.at[idx_v.at[pl.ds(off, chunk)]],
                             rows[b], sems[b])

        def wait_gather(b):
            pltpu.make_async_copy(x_hbm.at[pl.ds(0, chunk)], rows[b],
                                  sems[b]).wait()

        start_gather(0, 0)
        start_gather(1, 1)

        def body(jj, carry):
            for b in range(2):
                j = jj * 2 + b
                wait_gather(b)
                off = pl.multiple_of(base_w + j * chunk, 8)
                pltpu.sync_copy(rows[b], out_hbm.at[pl.ds(off, chunk)])

                @pl.when(j + 2 < nch_w)
                def _():
                    start_gather(j + 2, b)

            return carry

        lax.fori_loop(0, nch_w // 2, body, 0)

    return shuffle_k


def kernel(X):
    n, d = X.shape
    perm = jnp.asarray(_perm_const(n))
    return _make_gather(n, d)(X, perm)
